# baseline (speedup 1.0000x reference)
import functools

import jax
import jax.numpy as jnp
from jax import lax
from jax.experimental import pallas as pl
from jax.experimental.pallas import tpu as pltpu
from jax.experimental.pallas import tpu_sc as plsc

DIM = 100
PADW = 128
SEQ = 200
SPLIT = 104
NBUF = 4


@functools.cache
def _make_kernel(nb, seq):
    assert seq == SEQ
    info = plsc.get_sparse_core_info()
    nc, ns = info.num_cores, info.num_subcores
    nw = nc * ns
    n_chunks = nb // nw
    assert nb % nw == 0 and n_chunks % NBUF == 0 and n_chunks >= 2 * NBUF

    def body(x_hbm, table_hbm, out_hbm, *refs):
        idx_b = refs[0:NBUF]
        rows_p = refs[NBUF:2 * NBUF]
        sem_i = refs[2 * NBUF:3 * NBUF]
        sem_g = refs[3 * NBUF:4 * NBUF]
        sem_o = refs[4 * NBUF:5 * NBUF]
        wid = lax.axis_index("s") * nc + lax.axis_index("c")
        base = wid * n_chunks

        def start_idx(c, b):
            pltpu.async_copy(x_hbm.at[base + c], idx_b[b], sem_i[b])

        def wait_idx(b):
            pltpu.make_async_copy(x_hbm.at[base], idx_b[b], sem_i[b]).wait()

        def start_gather(b):
            pltpu.async_copy(
                table_hbm.at[idx_b[b].at[pl.ds(0, SPLIT)]],
                rows_p[b].at[pl.ds(0, SPLIT)],
                sem_g[b],
            )
            pltpu.async_copy(
                table_hbm.at[idx_b[b].at[pl.ds(SPLIT, SEQ - SPLIT)]],
                rows_p[b].at[pl.ds(SPLIT, SEQ - SPLIT)],
                sem_g[b],
            )

        def wait_gather(b):
            pltpu.make_async_copy(
                table_hbm.at[idx_b[b].at[pl.ds(0, SPLIT)]],
                rows_p[b].at[pl.ds(0, SPLIT)],
                sem_g[b],
            ).wait()
            pltpu.make_async_copy(
                table_hbm.at[idx_b[b].at[pl.ds(SPLIT, SEQ - SPLIT)]],
                rows_p[b].at[pl.ds(SPLIT, SEQ - SPLIT)],
                sem_g[b],
            ).wait()

        def start_out(c, b):
            pltpu.async_copy(rows_p[b], out_hbm.at[base + c], sem_o[b])

        def wait_out(b):
            pltpu.make_async_copy(rows_p[b], out_hbm.at[base], sem_o[b]).wait()

        def handle(c, b, start_next_idx, start_next_gather, prior_out):
            wait_gather(b)
            start_out(c, b)
            if start_next_idx:
                start_idx(c + NBUF, b)
            if start_next_gather:
                nb3 = (b + NBUF - 1) % NBUF
                wait_idx(nb3)
                if prior_out:
                    wait_out(nb3)
                start_gather(nb3)

        for k in range(NBUF):
            start_idx(k, k)
        for k in range(NBUF - 1):
            wait_idx(k)
            start_gather(k)

        handle(0, 0, True, True, False)
        for c in range(1, NBUF):
            handle(c, c % NBUF, True, True, True)

        def outer(g, carry):
            c0 = g * NBUF
            for u in range(NBUF):
                handle(c0 + u, u, True, True, True)
            return carry

        lax.fori_loop(1, n_chunks // NBUF - 1, outer, 0)

        for u in range(NBUF):
            c = n_chunks - NBUF + u
            handle(c, u, False, u == 0, u == 0)
        for u in range(NBUF):
            wait_out(u)

    mesh = plsc.VectorSubcoreMesh(core_axis_name="c", subcore_axis_name="s")
    return pl.kernel(
        body,
        out_type=jax.ShapeDtypeStruct((nb, SEQ, PADW), jnp.float32),
        mesh=mesh,
        compiler_params=pltpu.CompilerParams(
            use_tc_tiling_on_sc=True, needs_layout_passes=False
        ),
        scratch_types=(
            [pltpu.VMEM((SEQ,), jnp.int32) for _ in range(NBUF)]
            + [pltpu.VMEM((SEQ, PADW), jnp.float32) for _ in range(NBUF)]
            + [pltpu.SemaphoreType.DMA] * (3 * NBUF)
        ),
    )


def kernel(x, table):
    nb, seq = x.shape
    xi = x.astype(jnp.int32)
    tp = jnp.pad(table, ((0, 0), (0, PADW - DIM)))
    out = _make_kernel(nb, seq)(xi, tp)
    return out[..., :DIM]


# speedup vs baseline: 1.0009x; 1.0009x over previous
import functools

import jax
import jax.numpy as jnp
from jax import lax
from jax.experimental import pallas as pl
from jax.experimental.pallas import tpu as pltpu
from jax.experimental.pallas import tpu_sc as plsc

DIM = 100
PADW = 128
SEQ = 200
SPLIT = 104
NBUF = 4


@functools.cache
def _make_kernel(nb, seq):
    assert seq == SEQ
    info = plsc.get_sparse_core_info()
    nc, ns = info.num_cores, info.num_subcores
    nw = nc * ns
    n_chunks = nb // nw
    assert nb % nw == 0 and n_chunks % NBUF == 0 and n_chunks >= 2 * NBUF

    def body(x_hbm, table_hbm, out_hbm, *refs):
        idx_b = refs[0:NBUF]
        rows_p = refs[NBUF:2 * NBUF]
        sem_i = refs[2 * NBUF:3 * NBUF]
        sem_g = refs[3 * NBUF:4 * NBUF]
        sem_o = refs[4 * NBUF:5 * NBUF]
        wid = lax.axis_index("s") * nc + lax.axis_index("c")
        base = wid * n_chunks

        def start_idx(c, b):
            pltpu.async_copy(x_hbm.at[base + c], idx_b[b], sem_i[b])

        def wait_idx(b):
            pltpu.make_async_copy(x_hbm.at[base], idx_b[b], sem_i[b]).wait()

        def start_gather(b):
            pltpu.async_copy(
                table_hbm.at[idx_b[b].at[pl.ds(0, SPLIT)]],
                rows_p[b].at[pl.ds(0, SPLIT)],
                sem_g[b],
            )
            pltpu.async_copy(
                table_hbm.at[idx_b[b].at[pl.ds(SPLIT, SEQ - SPLIT)]],
                rows_p[b].at[pl.ds(SPLIT, SEQ - SPLIT)],
                sem_g[b],
            )

        def wait_gather(b):
            pltpu.make_async_copy(
                table_hbm.at[idx_b[b].at[pl.ds(0, SPLIT)]],
                rows_p[b].at[pl.ds(0, SPLIT)],
                sem_g[b],
            ).wait()
            pltpu.make_async_copy(
                table_hbm.at[idx_b[b].at[pl.ds(SPLIT, SEQ - SPLIT)]],
                rows_p[b].at[pl.ds(SPLIT, SEQ - SPLIT)],
                sem_g[b],
            ).wait()

        def start_out(c, b):
            pltpu.async_copy(rows_p[b], out_hbm.at[base + c], sem_o[b])

        def wait_out(b):
            pltpu.make_async_copy(rows_p[b], out_hbm.at[base], sem_o[b]).wait()

        def handle(c, b, start_next_idx, start_next_gather, prior_out):
            wait_gather(b)
            if start_next_idx:
                start_idx(c + NBUF, b)
            if start_next_gather:
                nb3 = (b + NBUF - 1) % NBUF
                wait_idx(nb3)
                if prior_out:
                    wait_out(nb3)
                start_gather(nb3)
            start_out(c, b)

        for k in range(NBUF):
            start_idx(k, k)
        for k in range(NBUF - 1):
            wait_idx(k)
            start_gather(k)

        handle(0, 0, True, True, False)
        for c in range(1, NBUF):
            handle(c, c % NBUF, True, True, True)

        def outer(g, carry):
            c0 = g * NBUF
            for u in range(NBUF):
                handle(c0 + u, u, True, True, True)
            return carry

        lax.fori_loop(1, n_chunks // NBUF - 1, outer, 0)

        for u in range(NBUF):
            c = n_chunks - NBUF + u
            handle(c, u, False, u == 0, u == 0)
        for u in range(NBUF):
            wait_out(u)

    mesh = plsc.VectorSubcoreMesh(core_axis_name="c", subcore_axis_name="s")
    return pl.kernel(
        body,
        out_type=jax.ShapeDtypeStruct((nb, SEQ, PADW), jnp.float32),
        mesh=mesh,
        compiler_params=pltpu.CompilerParams(
            use_tc_tiling_on_sc=True, needs_layout_passes=False
        ),
        scratch_types=(
            [pltpu.VMEM((SEQ,), jnp.int32) for _ in range(NBUF)]
            + [pltpu.VMEM((SEQ, PADW), jnp.float32) for _ in range(NBUF)]
            + [pltpu.SemaphoreType.DMA] * (3 * NBUF)
        ),
    )


def kernel(x, table):
    nb, seq = x.shape
    xi = x.astype(jnp.int32)
    tp = jnp.pad(table, ((0, 0), (0, PADW - DIM)))
    out = _make_kernel(nb, seq)(xi, tp)
    return out[..., :DIM]
